# Initial kernel scaffold; baseline (speedup 1.0000x reference)
import jax
import jax.numpy as jnp
from jax.experimental import pallas as pl


def kernel(tp, sp, t, s, obs, alpha, beta):
    raise NotImplementedError("write your pallas kernel here")


# speedup vs baseline: 470.6700x; 470.6700x over previous
import functools

import jax
import jax.numpy as jnp
from jax import lax
from jax.experimental import pallas as pl
from jax.experimental.pallas import tpu as pltpu, tpu_sc as plsc

B = 1048576
N_T = 8
N_S = 8
TBL = N_T * N_S * N_T * N_S

NC = 2
NS = 16
L = 16
NW = NC * NS
PER_W = B // NW
CHUNK = 8192
N_CHUNKS = PER_W // CHUNK


def _make_sc_call():
    mesh = plsc.VectorSubcoreMesh(core_axis_name="c", subcore_axis_name="s")

    @functools.partial(
        pl.kernel,
        out_type=jax.ShapeDtypeStruct((B,), jnp.float32),
        mesh=mesh,
        compiler_params=pltpu.CompilerParams(needs_layout_passes=False),
        scratch_types=[
            pltpu.VMEM((TBL,), jnp.float32),
            pltpu.VMEM((N_T * N_S,), jnp.int32),
            pltpu.VMEM((N_S * N_S,), jnp.float32),
            pltpu.VMEM((L,), jnp.float32),
            pltpu.VMEM((CHUNK,), jnp.int32),
            pltpu.VMEM((CHUNK,), jnp.int32),
            pltpu.VMEM((CHUNK,), jnp.int32),
            pltpu.VMEM((CHUNK,), jnp.int32),
            pltpu.VMEM((CHUNK,), jnp.float32),
            pltpu.SemaphoreType.DMA,
        ],
    )
    def sc_kernel(tp_hbm, sp_hbm, t_hbm, s_hbm, obs_hbm, alpha_hbm, beta_hbm,
                  out_hbm,
                  table_v, obs_v, alpha_v, beta_v,
                  tp_v, sp_v, t_v, s_v, out_v, sem):
        wid = lax.axis_index("s") * NC + lax.axis_index("c")

        c1 = pltpu.async_copy(obs_hbm, obs_v, sem)
        c2 = pltpu.async_copy(alpha_hbm, alpha_v, sem)
        c3 = pltpu.async_copy(beta_hbm, beta_v, sem)
        c1.wait(); c2.wait(); c3.wait()

        beta = beta_v[...]
        lane = lax.iota(jnp.int32, L)

        def build(i, _):
            idx = i * L + lane
            tp_i = idx >> 9
            sp_i = (idx >> 6) & 7
            t_i = (idx >> 3) & 7
            s_i = idx & 7
            obs_g = plsc.load_gather(obs_v, [tp_i * N_S + sp_i]).astype(jnp.float32)
            al_g = plsc.load_gather(alpha_v, [sp_i * N_S + s_i])
            al_g = jnp.where(sp_i == s_i, al_g, 0.0)
            dt = jnp.abs(t_i - tp_i).astype(jnp.float32)
            table_v[pl.ds(i * L, L)] = al_g * obs_g * beta * jnp.exp(-beta * dt)
            return 0
        lax.fori_loop(0, TBL // L, build, 0)

        def step(c, _):
            base = wid * PER_W + c * CHUNK
            d1 = pltpu.async_copy(tp_hbm.at[pl.ds(base, CHUNK)], tp_v, sem)
            d2 = pltpu.async_copy(sp_hbm.at[pl.ds(base, CHUNK)], sp_v, sem)
            d3 = pltpu.async_copy(t_hbm.at[pl.ds(base, CHUNK)], t_v, sem)
            d4 = pltpu.async_copy(s_hbm.at[pl.ds(base, CHUNK)], s_v, sem)
            d1.wait(); d2.wait(); d3.wait(); d4.wait()

            def body(k, _):
                sl = pl.ds(k * L, L)
                flat = ((tp_v[sl] * N_S + sp_v[sl]) * N_T + t_v[sl]) * N_S + s_v[sl]
                out_v[sl] = plsc.load_gather(table_v, [flat])
                return 0
            lax.fori_loop(0, CHUNK // L, body, 0)

            pltpu.sync_copy(out_v, out_hbm.at[pl.ds(base, CHUNK)])
            return 0
        lax.fori_loop(0, N_CHUNKS, step, 0)

    return sc_kernel


_SC_CALL = _make_sc_call()


def kernel(tp, sp, t, s, obs, alpha, beta):
    obs_f = obs.reshape(-1).astype(jnp.int32)
    alpha_f = alpha.reshape(-1).astype(jnp.float32)
    beta16 = jnp.broadcast_to(beta.astype(jnp.float32), (L,))
    return _SC_CALL(tp, sp, t, s, obs_f, alpha_f, beta16)


# speedup vs baseline: 595.1220x; 1.2644x over previous
import jax
import jax.numpy as jnp
from jax import lax
from jax.experimental import pallas as pl
from jax.experimental.pallas import tpu as pltpu, tpu_sc as plsc

B = 1048576
N_T = 8
N_S = 8
TBL = N_T * N_S * N_T * N_S

NC = 2
NS = 16
L = 16
NW = NC * NS
PER_W = B // NW
CHUNK = 8192
N_CHUNKS = PER_W // CHUNK


def _make_sc_call():
    mesh = plsc.VectorSubcoreMesh(core_axis_name="c", subcore_axis_name="s")

    chunk_i32 = pltpu.VMEM((CHUNK,), jnp.int32)
    chunk_f32 = pltpu.VMEM((CHUNK,), jnp.float32)

    def sc_kernel(tp_hbm, sp_hbm, t_hbm, s_hbm, obs_hbm, alpha_hbm, beta_hbm,
                  out_hbm,
                  table_v, obs_v, alpha_v, beta_v,
                  tp0, sp0, t0, s0, o0,
                  tp1, sp1, t1, s1, o1,
                  sem_in, sem_out):
        wid = lax.axis_index("s") * NC + lax.axis_index("c")

        c1 = pltpu.async_copy(obs_hbm, obs_v, sem_in)
        c2 = pltpu.async_copy(alpha_hbm, alpha_v, sem_in)
        c3 = pltpu.async_copy(beta_hbm, beta_v, sem_in)
        c1.wait(); c2.wait(); c3.wait()

        beta = beta_v[...]
        lane = lax.iota(jnp.int32, L)

        @plsc.parallel_loop(0, TBL // L, unroll=4)
        def _build(i):
            idx = i * L + lane
            tp_i = idx >> 9
            sp_i = (idx >> 6) & 7
            t_i = (idx >> 3) & 7
            s_i = idx & 7
            obs_g = plsc.load_gather(obs_v, [tp_i * N_S + sp_i]).astype(jnp.float32)
            al_g = plsc.load_gather(alpha_v, [sp_i * N_S + s_i])
            al_g = jnp.where(sp_i == s_i, al_g, 0.0)
            dt = jnp.abs(t_i - tp_i).astype(jnp.float32)
            table_v[pl.ds(i * L, L)] = al_g * obs_g * beta * jnp.exp(-beta * dt)

        banks = ((tp0, sp0, t0, s0, o0), (tp1, sp1, t1, s1, o1))
        srcs = (tp_hbm, sp_hbm, t_hbm, s_hbm)

        def issue_in(c, bufs):
            base = wid * PER_W + c * CHUNK
            return [pltpu.async_copy(src.at[pl.ds(base, CHUNK)], dst, sem_in)
                    for src, dst in zip(srcs, bufs)]

        out_copies = [None, None]
        cps = issue_in(0, banks[0][:4])
        for c in range(N_CHUNKS):
            if c + 1 < N_CHUNKS:
                next_cps = issue_in(c + 1, banks[(c + 1) % 2][:4])
            for cp in cps:
                cp.wait()
            if out_copies[c % 2] is not None:
                out_copies[c % 2].wait()

            tp_v, sp_v, t_v, s_v, out_v = banks[c % 2]

            @plsc.parallel_loop(0, CHUNK // L, unroll=8)
            def _gather(k):
                sl = pl.ds(k * L, L)
                flat = ((tp_v[sl] * N_S + sp_v[sl]) * N_T + t_v[sl]) * N_S + s_v[sl]
                out_v[sl] = plsc.load_gather(table_v, [flat])

            base = wid * PER_W + c * CHUNK
            out_copies[c % 2] = pltpu.async_copy(
                out_v, out_hbm.at[pl.ds(base, CHUNK)], sem_out)
            if c + 1 < N_CHUNKS:
                cps = next_cps
        for oc in out_copies:
            if oc is not None:
                oc.wait()

    return pl.kernel(
        sc_kernel,
        out_type=jax.ShapeDtypeStruct((B,), jnp.float32),
        mesh=mesh,
        compiler_params=pltpu.CompilerParams(needs_layout_passes=False),
        scratch_types=[
            pltpu.VMEM((TBL,), jnp.float32),
            pltpu.VMEM((N_T * N_S,), jnp.int32),
            pltpu.VMEM((N_S * N_S,), jnp.float32),
            pltpu.VMEM((L,), jnp.float32),
            chunk_i32, chunk_i32, chunk_i32, chunk_i32, chunk_f32,
            chunk_i32, chunk_i32, chunk_i32, chunk_i32, chunk_f32,
            pltpu.SemaphoreType.DMA,
            pltpu.SemaphoreType.DMA,
        ],
    )


_SC_CALL = _make_sc_call()


def kernel(tp, sp, t, s, obs, alpha, beta):
    obs_f = obs.reshape(-1).astype(jnp.int32)
    alpha_f = alpha.reshape(-1).astype(jnp.float32)
    beta16 = jnp.broadcast_to(beta.astype(jnp.float32), (L,))
    return _SC_CALL(tp, sp, t, s, obs_f, alpha_f, beta16)
